# baseline (speedup 1.0000x reference)
import functools

import jax
import jax.numpy as jnp
from jax import lax
from jax.experimental import pallas as pl
from jax.experimental.pallas import tpu as pltpu
from jax.experimental.pallas import tpu_sc as plsc

NUM_CODES = 1024
D = 64
BETA = 0.25

B0 = 8
B1 = 4608
N_ROWS = B0 * B1
HALF = N_ROWS // 2

NC = 2
NS = 16
NW = NC * NS

DPAD = 128
LW = 768


def _dist_argmin_kernel(z_ref, cm2_ref, idx_ref, loss_ref, csq_ref, zc_ref):
    i = pl.program_id(0)
    cm2 = cm2_ref[...]

    @pl.when(i == 0)
    def _():
        csq_ref[...] = jnp.sum(cm2 * cm2, axis=1, keepdims=True) * 0.25
        loss_ref[...] = jnp.zeros((1, 1), jnp.float32)

    z = z_ref[...]
    zc_ref[...] = lax.dot_general(cm2, z, (((1,), (1,)), ((), ())),
                                  preferred_element_type=jnp.float32)
    z_sq = jnp.sum(z * z, axis=1, keepdims=True).T

    for t in range(B1 // LW):
        zs = z_sq[:, t * LW:(t + 1) * LW]

        def body(r, carry):
            v, bi = carry
            d = (zs + zc_ref[pl.ds(r * 8, 8), pl.ds(t * LW, LW)]) \
                + csq_ref[pl.ds(r * 8, 8), :]
            better = d < v
            bi = jnp.where(better, r, bi)
            v = jnp.minimum(v, d)
            return v, bi

        v0 = jnp.full((8, LW), jnp.inf, jnp.float32)
        b0 = jnp.zeros((8, LW), jnp.int32)
        v, bi = lax.fori_loop(0, NUM_CODES // 8, body, (v0, b0))
        m = jnp.min(v, axis=0, keepdims=True)
        srow = lax.broadcasted_iota(jnp.int32, (8, LW), 0)
        cand = bi * 8 + srow
        idx = jnp.min(jnp.where(v == m, cand, NUM_CODES), axis=0)
        idx_ref[0, 0, pl.ds(t * LW, LW)] = idx
        loss_ref[...] += jnp.sum(m).reshape(1, 1)


def _tc_dist_argmin(z2, codebook_m2):
    nblk = z2.shape[0] // B1
    idx, loss_sum = pl.pallas_call(
        _dist_argmin_kernel,
        grid=(nblk,),
        in_specs=[
            pl.BlockSpec((B1, D), lambda i: (i, 0)),
            pl.BlockSpec((NUM_CODES, D), lambda i: (0, 0)),
        ],
        out_specs=[
            pl.BlockSpec((1, 1, B1), lambda i: (i, 0, 0)),
            pl.BlockSpec((1, 1), lambda i: (0, 0)),
        ],
        out_shape=[
            jax.ShapeDtypeStruct((nblk, 1, B1), jnp.int32),
            jax.ShapeDtypeStruct((1, 1), jnp.float32),
        ],
        scratch_shapes=[pltpu.VMEM((NUM_CODES, 1), jnp.float32),
                        pltpu.VMEM((NUM_CODES, B1), jnp.float32)],
    )(z2, codebook_m2)
    return idx.reshape(z2.shape[0]), loss_sum


CHUNK = 576


def _sc_gather(codebook_padded, idx_flat):
    n = idx_flat.shape[0]
    b_per_w = n // NW
    mesh = plsc.VectorSubcoreMesh(core_axis_name="c", subcore_axis_name="s")

    @functools.partial(
        pl.kernel,
        mesh=mesh,
        out_type=jax.ShapeDtypeStruct((n, DPAD), jnp.float32),
        scratch_types=[
            pltpu.VMEM((CHUNK,), jnp.int32),
            pltpu.VMEM((CHUNK, DPAD), jnp.float32),
            pltpu.SemaphoreType.DMA,
        ],
    )
    def k(table_hbm, idx_hbm, out_hbm, idx_v, rows_v, sem):
        wid = lax.axis_index("s") * NC + lax.axis_index("c")
        base = wid * b_per_w

        @pl.loop(0, b_per_w // CHUNK)
        def _(j):
            b = base + j * CHUNK
            pltpu.sync_copy(idx_hbm.at[pl.ds(b, CHUNK)], idx_v)
            pltpu.async_copy(table_hbm.at[idx_v], rows_v, sem).wait()
            pltpu.sync_copy(rows_v, out_hbm.at[pl.ds(b, CHUNK)])

    return k(codebook_padded, idx_flat)


def kernel(z, codebook):
    codebook_m2 = codebook * -2.0
    codebook_padded = jnp.pad(codebook, ((0, 0), (0, DPAD - D)))
    z2 = z.reshape(N_ROWS, D)
    idx, loss_sum = _tc_dist_argmin(z2, codebook_m2)
    z_q = _sc_gather(codebook_padded, idx)[:, :D]
    loss = loss_sum[0, 0] * (2.0 * BETA / (N_ROWS * D))
    return z_q.reshape(z.shape), loss


# speedup vs baseline: 3.9075x; 3.9075x over previous
import functools

import jax
import jax.numpy as jnp
from jax import lax
from jax.experimental import pallas as pl
from jax.experimental.pallas import tpu as pltpu
from jax.experimental.pallas import tpu_sc as plsc

NUM_CODES = 1024
D = 64
BETA = 0.25

B0 = 8
B1 = 4608
N_ROWS = B0 * B1
HALF = N_ROWS // 2

NC = 2
NS = 16
NW = NC * NS

DPAD = 128
LW = 768


def _dist_argmin_kernel(z_ref, cm2_ref, idx_ref, loss_ref, csq_ref, zc_ref):
    i = pl.program_id(0)
    cm2 = cm2_ref[...]

    @pl.when(i == 0)
    def _():
        csq_ref[...] = jnp.sum(cm2 * cm2, axis=1, keepdims=True) * 0.25
        loss_ref[...] = jnp.zeros((1, 1), jnp.float32)

    z = z_ref[...]
    zc_ref[...] = lax.dot_general(cm2, z, (((1,), (1,)), ((), ())),
                                  preferred_element_type=jnp.float32)
    z_sq = jnp.sum(z * z, axis=1, keepdims=True).T

    for t in range(B1 // LW):
        zs = z_sq[:, t * LW:(t + 1) * LW]

        def body(r, carry):
            v, bi = carry
            d = (zs + zc_ref[pl.ds(r * 8, 8), pl.ds(t * LW, LW)]) \
                + csq_ref[pl.ds(r * 8, 8), :]
            better = d < v
            bi = jnp.where(better, r, bi)
            v = jnp.minimum(v, d)
            return v, bi

        v0 = jnp.full((8, LW), jnp.inf, jnp.float32)
        b0 = jnp.zeros((8, LW), jnp.int32)
        v, bi = lax.fori_loop(0, NUM_CODES // 8, body, (v0, b0),
                              unroll=True)
        m = jnp.min(v, axis=0, keepdims=True)
        srow = lax.broadcasted_iota(jnp.int32, (8, LW), 0)
        cand = bi * 8 + srow
        idx = jnp.min(jnp.where(v == m, cand, NUM_CODES), axis=0)
        idx_ref[0, 0, pl.ds(t * LW, LW)] = idx
        loss_ref[...] += jnp.sum(m).reshape(1, 1)


def _tc_dist_argmin(z2, codebook_m2):
    nblk = z2.shape[0] // B1
    idx, loss_sum = pl.pallas_call(
        _dist_argmin_kernel,
        grid=(nblk,),
        in_specs=[
            pl.BlockSpec((B1, D), lambda i: (i, 0)),
            pl.BlockSpec((NUM_CODES, D), lambda i: (0, 0)),
        ],
        out_specs=[
            pl.BlockSpec((1, 1, B1), lambda i: (i, 0, 0)),
            pl.BlockSpec((1, 1), lambda i: (0, 0)),
        ],
        out_shape=[
            jax.ShapeDtypeStruct((nblk, 1, B1), jnp.int32),
            jax.ShapeDtypeStruct((1, 1), jnp.float32),
        ],
        scratch_shapes=[pltpu.VMEM((NUM_CODES, 1), jnp.float32),
                        pltpu.VMEM((NUM_CODES, B1), jnp.float32)],
    )(z2, codebook_m2)
    return idx.reshape(z2.shape[0]), loss_sum


CHUNK = 576


def _sc_gather(codebook_padded, idx_flat):
    n = idx_flat.shape[0]
    b_per_w = n // NW
    mesh = plsc.VectorSubcoreMesh(core_axis_name="c", subcore_axis_name="s")

    @functools.partial(
        pl.kernel,
        mesh=mesh,
        out_type=jax.ShapeDtypeStruct((n, DPAD), jnp.float32),
        scratch_types=[
            pltpu.VMEM((CHUNK,), jnp.int32),
            pltpu.VMEM((CHUNK, DPAD), jnp.float32),
            pltpu.SemaphoreType.DMA,
        ],
    )
    def k(table_hbm, idx_hbm, out_hbm, idx_v, rows_v, sem):
        wid = lax.axis_index("s") * NC + lax.axis_index("c")
        base = wid * b_per_w

        @pl.loop(0, b_per_w // CHUNK)
        def _(j):
            b = base + j * CHUNK
            pltpu.sync_copy(idx_hbm.at[pl.ds(b, CHUNK)], idx_v)
            pltpu.async_copy(table_hbm.at[idx_v], rows_v, sem).wait()
            pltpu.sync_copy(rows_v, out_hbm.at[pl.ds(b, CHUNK)])

    return k(codebook_padded, idx_flat)


def kernel(z, codebook):
    codebook_m2 = codebook * -2.0
    codebook_padded = jnp.pad(codebook, ((0, 0), (0, DPAD - D)))
    z2 = z.reshape(N_ROWS, D)
    idx, loss_sum = _tc_dist_argmin(z2, codebook_m2)
    z_q = _sc_gather(codebook_padded, idx)[:, :D]
    loss = loss_sum[0, 0] * (2.0 * BETA / (N_ROWS * D))
    return z_q.reshape(z.shape), loss


# speedup vs baseline: 4.0845x; 1.0453x over previous
import functools

import jax
import jax.numpy as jnp
from jax import lax
from jax.experimental import pallas as pl
from jax.experimental.pallas import tpu as pltpu
from jax.experimental.pallas import tpu_sc as plsc

NUM_CODES = 1024
D = 64
BETA = 0.25

B0 = 8
B1 = 4608
N_ROWS = B0 * B1
HALF = N_ROWS // 2

NC = 2
NS = 16
NW = NC * NS

DPAD = 128
LW = 2304


def _dist_argmin_kernel(z_ref, cm2_ref, idx_ref, loss_ref, csq_ref, zc_ref):
    i = pl.program_id(0)
    cm2 = cm2_ref[...]

    @pl.when(i == 0)
    def _():
        csq_ref[...] = jnp.sum(cm2 * cm2, axis=1, keepdims=True) * 0.25
        loss_ref[...] = jnp.zeros((1, 1), jnp.float32)

    z = z_ref[...]
    zc_ref[...] = lax.dot_general(cm2, z, (((1,), (1,)), ((), ())),
                                  preferred_element_type=jnp.float32)
    z_sq = jnp.sum(z * z, axis=1, keepdims=True).T

    for t in range(B1 // LW):
        zs = z_sq[:, t * LW:(t + 1) * LW]

        def body(r, carry):
            v, bi = carry
            d = (zs + zc_ref[pl.ds(r * 8, 8), pl.ds(t * LW, LW)]) \
                + csq_ref[pl.ds(r * 8, 8), :]
            better = d < v
            bi = jnp.where(better, r, bi)
            v = jnp.minimum(v, d)
            return v, bi

        v0 = jnp.full((8, LW), jnp.inf, jnp.float32)
        b0 = jnp.zeros((8, LW), jnp.int32)
        v, bi = lax.fori_loop(0, NUM_CODES // 8, body, (v0, b0),
                              unroll=True)
        m = jnp.min(v, axis=0, keepdims=True)
        srow = lax.broadcasted_iota(jnp.int32, (8, LW), 0)
        cand = bi * 8 + srow
        idx = jnp.min(jnp.where(v == m, cand, NUM_CODES), axis=0)
        idx_ref[0, 0, pl.ds(t * LW, LW)] = idx
        loss_ref[...] += jnp.sum(m).reshape(1, 1)


def _tc_dist_argmin(z2, codebook_m2):
    nblk = z2.shape[0] // B1
    idx, loss_sum = pl.pallas_call(
        _dist_argmin_kernel,
        grid=(nblk,),
        in_specs=[
            pl.BlockSpec((B1, D), lambda i: (i, 0)),
            pl.BlockSpec((NUM_CODES, D), lambda i: (0, 0)),
        ],
        out_specs=[
            pl.BlockSpec((1, 1, B1), lambda i: (i, 0, 0)),
            pl.BlockSpec((1, 1), lambda i: (0, 0)),
        ],
        out_shape=[
            jax.ShapeDtypeStruct((nblk, 1, B1), jnp.int32),
            jax.ShapeDtypeStruct((1, 1), jnp.float32),
        ],
        scratch_shapes=[pltpu.VMEM((NUM_CODES, 1), jnp.float32),
                        pltpu.VMEM((NUM_CODES, B1), jnp.float32)],
    )(z2, codebook_m2)
    return idx.reshape(z2.shape[0]), loss_sum


CHUNK = 576


def _sc_gather(codebook_padded, idx_flat):
    n = idx_flat.shape[0]
    b_per_w = n // NW
    mesh = plsc.VectorSubcoreMesh(core_axis_name="c", subcore_axis_name="s")

    @functools.partial(
        pl.kernel,
        mesh=mesh,
        out_type=jax.ShapeDtypeStruct((n, DPAD), jnp.float32),
        scratch_types=[
            pltpu.VMEM((CHUNK,), jnp.int32),
            pltpu.VMEM((CHUNK, DPAD), jnp.float32),
            pltpu.SemaphoreType.DMA,
        ],
    )
    def k(table_hbm, idx_hbm, out_hbm, idx_v, rows_v, sem):
        wid = lax.axis_index("s") * NC + lax.axis_index("c")
        base = wid * b_per_w

        @pl.loop(0, b_per_w // CHUNK)
        def _(j):
            b = base + j * CHUNK
            pltpu.sync_copy(idx_hbm.at[pl.ds(b, CHUNK)], idx_v)
            pltpu.async_copy(table_hbm.at[idx_v], rows_v, sem).wait()
            pltpu.sync_copy(rows_v, out_hbm.at[pl.ds(b, CHUNK)])

    return k(codebook_padded, idx_flat)


def kernel(z, codebook):
    codebook_m2 = codebook * -2.0
    codebook_padded = jnp.pad(codebook, ((0, 0), (0, DPAD - D)))
    z2 = z.reshape(N_ROWS, D)
    idx, loss_sum = _tc_dist_argmin(z2, codebook_m2)
    z_q = _sc_gather(codebook_padded, idx)[:, :D]
    loss = loss_sum[0, 0] * (2.0 * BETA / (N_ROWS * D))
    return z_q.reshape(z.shape), loss
